# baseline (speedup 1.0000x reference)
import jax
import jax.numpy as jnp
from jax import lax
from jax.experimental import pallas as pl
from jax.experimental.pallas import tpu as pltpu
from jax.experimental.pallas import tpu_sc as plsc

_B, _N, _F = 1024, 20, 19
_V = 101
_C = 128
_NW = 32
_PER_W = _B // _NW
_IMG = _C * _N * _N


def _sc_body(idx_hbm, emb_hbm, out_hbm, idxs_v, rows_v, buf0, buf1,
             gsem, s0, s1):
    wid = lax.axis_index("s") * 2 + lax.axis_index("c")
    base = wid * _PER_W

    pltpu.make_async_copy(
        idx_hbm.at[pl.ds(base, _PER_W)], idxs_v, gsem).start()
    pltpu.make_async_copy(
        idx_hbm.at[pl.ds(base, _PER_W)], idxs_v, gsem).wait()

    zero16 = jnp.zeros((16,), jnp.float32)

    def zbody(i, carry):
        off = pl.multiple_of(i * 128, 128)
        for k in range(8):
            buf0[pl.ds(off + k * 16, 16)] = zero16
            buf1[pl.ds(off + k * 16, 16)] = zero16
        return carry

    lax.fori_loop(0, _IMG // 128, zbody, 0)

    iota = lax.iota(jnp.int32, 16)

    def fill(buf, bl):
        pltpu.make_async_copy(emb_hbm.at[idxs_v.at[bl]], rows_v, gsem).start()
        pltpu.make_async_copy(emb_hbm.at[idxs_v.at[bl]], rows_v, gsem).wait()
        for k in range(_C // 16):
            qk = (16 * k + iota) * (_N * _N)
            for n in range(_N):
                val = rows_v[n, pl.ds(k * 16, 16)]
                plsc.store_scatter(buf, [qk + (_N + 1) * n], val)

    def lbody(g, carry):
        b0 = g * 2
        b1 = b0 + 1

        @pl.when(g > 0)
        def _():
            pltpu.make_async_copy(buf0, out_hbm.at[base + b0 - 2], s0).wait()

        fill(buf0, b0)
        pltpu.make_async_copy(buf0, out_hbm.at[base + b0], s0).start()

        @pl.when(g > 0)
        def _():
            pltpu.make_async_copy(buf1, out_hbm.at[base + b1 - 2], s1).wait()

        fill(buf1, b1)
        pltpu.make_async_copy(buf1, out_hbm.at[base + b1], s1).start()
        return carry

    lax.fori_loop(0, _PER_W // 2, lbody, 0)

    pltpu.make_async_copy(buf0, out_hbm.at[base + _PER_W - 2], s0).wait()
    pltpu.make_async_copy(buf1, out_hbm.at[base + _PER_W - 1], s1).wait()


def kernel(batch_node_attr, emb_table):
    idx = batch_node_attr[:, :, 0].astype(jnp.int32)
    run = pl.kernel(
        _sc_body,
        out_type=jax.ShapeDtypeStruct((_B, _IMG), jnp.float32),
        mesh=plsc.VectorSubcoreMesh(core_axis_name="c", subcore_axis_name="s"),
        compiler_params=pltpu.CompilerParams(needs_layout_passes=False),
        scratch_types=[
            pltpu.VMEM((_PER_W, _N), jnp.int32),
            pltpu.VMEM((_N, _C), jnp.float32),
            pltpu.VMEM((_IMG,), jnp.float32),
            pltpu.VMEM((_IMG,), jnp.float32),
            pltpu.SemaphoreType.DMA,
            pltpu.SemaphoreType.DMA,
            pltpu.SemaphoreType.DMA,
        ],
    )
    out = run(idx, emb_table)
    return out.reshape(_B, _C, _N, _N)


# speedup vs baseline: 1.7984x; 1.7984x over previous
import jax
import jax.numpy as jnp
from jax import lax
from jax.experimental import pallas as pl
from jax.experimental.pallas import tpu as pltpu
from jax.experimental.pallas import tpu_sc as plsc

_B, _N, _F = 1024, 20, 19
_V = 101
_C = 128
_NW = 32
_PER_W = _B // _NW
_HC = _C // 2
_NN = _N * _N


def _sc_body(idx_hbm, emb_hbm, out_hbm, idxs_v, rows_v, buf0, buf1,
             gsem, s0, s1):
    wid = lax.axis_index("s") * 2 + lax.axis_index("c")
    base = wid * _PER_W

    pltpu.make_async_copy(
        idx_hbm.at[pl.ds(base, _PER_W)], idxs_v, gsem).start()
    pltpu.make_async_copy(
        idx_hbm.at[pl.ds(base, _PER_W)], idxs_v, gsem).wait()

    zero16 = jnp.zeros((16,), jnp.float32)

    def zbody(c, carry):
        for k in range(_NN // 16):
            buf0[c, pl.ds(k * 16, 16)] = zero16
            buf1[c, pl.ds(k * 16, 16)] = zero16
        return carry

    lax.fori_loop(0, _HC, zbody, 0)

    iota = lax.iota(jnp.int32, 16)

    def fill(buf, c0):
        for k in range(_HC // 16):
            cvec = 16 * k + iota
            for n in range(_N):
                val = rows_v[n, pl.ds(c0 + k * 16, 16)]
                colv = iota * 0 + (_N + 1) * n
                plsc.store_scatter(buf, [cvec, colv], val)

    def lbody(g, carry):
        pltpu.make_async_copy(emb_hbm.at[idxs_v.at[g]], rows_v, gsem).start()
        pltpu.make_async_copy(emb_hbm.at[idxs_v.at[g]], rows_v, gsem).wait()

        @pl.when(g > 0)
        def _():
            pltpu.make_async_copy(
                buf0, out_hbm.at[base + g - 1, pl.ds(0, _HC)], s0).wait()

        fill(buf0, 0)
        pltpu.make_async_copy(
            buf0, out_hbm.at[base + g, pl.ds(0, _HC)], s0).start()

        @pl.when(g > 0)
        def _():
            pltpu.make_async_copy(
                buf1, out_hbm.at[base + g - 1, pl.ds(_HC, _HC)], s1).wait()

        fill(buf1, _HC)
        pltpu.make_async_copy(
            buf1, out_hbm.at[base + g, pl.ds(_HC, _HC)], s1).start()
        return carry

    lax.fori_loop(0, _PER_W, lbody, 0)

    pltpu.make_async_copy(
        buf0, out_hbm.at[base + _PER_W - 1, pl.ds(0, _HC)], s0).wait()
    pltpu.make_async_copy(
        buf1, out_hbm.at[base + _PER_W - 1, pl.ds(_HC, _HC)], s1).wait()


def kernel(batch_node_attr, emb_table):
    idx = batch_node_attr[:, :, 0].astype(jnp.int32)
    run = pl.kernel(
        _sc_body,
        out_type=jax.ShapeDtypeStruct((_B, _C, _NN), jnp.float32),
        mesh=plsc.VectorSubcoreMesh(core_axis_name="c", subcore_axis_name="s"),
        compiler_params=pltpu.CompilerParams(
            needs_layout_passes=False, use_tc_tiling_on_sc=True),
        scratch_types=[
            pltpu.VMEM((_PER_W, _N), jnp.int32),
            pltpu.VMEM((_N, _C), jnp.float32),
            pltpu.VMEM((_HC, _NN), jnp.float32),
            pltpu.VMEM((_HC, _NN), jnp.float32),
            pltpu.SemaphoreType.DMA,
            pltpu.SemaphoreType.DMA,
            pltpu.SemaphoreType.DMA,
        ],
    )
    out = run(idx, emb_table)
    return out.reshape(_B, _C, _N, _N)


# speedup vs baseline: 1.9149x; 1.0647x over previous
import jax
import jax.numpy as jnp
from jax import lax
from jax.experimental import pallas as pl
from jax.experimental.pallas import tpu as pltpu
from jax.experimental.pallas import tpu_sc as plsc

_B, _N, _F = 1024, 20, 19
_V = 101
_C = 128
_NW = 32
_PER_W = _B // _NW
_ROWS_W = _PER_W * _N
_NN = _N * _N
_BB = 16


def _sc_gather_body(idx_hbm, emb_hbm, g_hbm, idxs_v, rows_v, gsem, osem):
    wid = lax.axis_index("s") * 2 + lax.axis_index("c")
    base = wid * _ROWS_W

    pltpu.make_async_copy(
        idx_hbm.at[pl.ds(base, _ROWS_W)], idxs_v, gsem).start()
    pltpu.make_async_copy(
        idx_hbm.at[pl.ds(base, _ROWS_W)], idxs_v, gsem).wait()

    pltpu.make_async_copy(emb_hbm.at[idxs_v], rows_v, gsem).start()
    pltpu.make_async_copy(emb_hbm.at[idxs_v], rows_v, gsem).wait()

    pltpu.make_async_copy(rows_v, g_hbm.at[pl.ds(base, _ROWS_W)], osem).start()
    pltpu.make_async_copy(rows_v, g_hbm.at[pl.ds(base, _ROWS_W)], osem).wait()


def _diag_expand_kernel(g_ref, out_ref):
    n_iota = lax.broadcasted_iota(jnp.int32, (_N, _NN), 0)
    j_iota = lax.broadcasted_iota(jnp.int32, (_N, _NN), 1)
    sel = (j_iota == (_N + 1) * n_iota).astype(jnp.float32)
    for b in range(_BB):
        gb = g_ref[pl.ds(b * _N, _N), :]
        out_ref[b] = lax.dot_general(
            gb, sel, (((0,), (0,)), ((), ())),
            preferred_element_type=jnp.float32)


def kernel(batch_node_attr, emb_table):
    idx = batch_node_attr[:, :, 0].astype(jnp.int32).reshape(_B * _N)

    gather = pl.kernel(
        _sc_gather_body,
        out_type=jax.ShapeDtypeStruct((_B * _N, _C), jnp.float32),
        mesh=plsc.VectorSubcoreMesh(core_axis_name="c", subcore_axis_name="s"),
        compiler_params=pltpu.CompilerParams(needs_layout_passes=False),
        scratch_types=[
            pltpu.VMEM((_ROWS_W,), jnp.int32),
            pltpu.VMEM((_ROWS_W, _C), jnp.float32),
            pltpu.SemaphoreType.DMA,
            pltpu.SemaphoreType.DMA,
        ],
    )
    g = gather(idx, emb_table)

    out = pl.pallas_call(
        _diag_expand_kernel,
        grid=(_B // _BB,),
        in_specs=[
            pl.BlockSpec((_BB * _N, _C), lambda i: (i, 0)),
        ],
        out_specs=pl.BlockSpec((_BB, _C, _NN), lambda i: (i, 0, 0)),
        out_shape=jax.ShapeDtypeStruct((_B, _C, _NN), jnp.float32),
    )(g)
    return out.reshape(_B, _C, _N, _N)
